# baseline (speedup 1.0000x reference)
import dataclasses

import numpy as np
import jax
import jax.numpy as jnp
from jax import lax
from jax.experimental import pallas as pl
from jax.experimental.pallas import tpu as pltpu
from jax.experimental.pallas import tpu_sc as plsc

B, L = 4096, 2048
BR = 64

_ERR_TABLE = np.array([
    [0.0072346226959187085, 0.000556539138304629],
    [0.007277966762969894, 0.0006341286519172679],
    [0.00751161242916645, 0.0003549648152118088],
    [0.003841922673385825, 0.0009420231103613149],
    [0.0028823807111780318, 0.00028250797805103186],
    [0.0025199820839086813, 0.0003671334743430223],
    [0.002398287605344948, 0.0002502935178378177],
    [0.0022981766524078407, 0.0002689046639815398],
    [0.0023829754751450204, 0.00029003694763449195],
    [0.002103783741402776, 0.00022834709513293414],
    [0.001680953479241577, 0.00017038730690437185],
    [0.0015808278649738481, 0.0001357746865960827],
    [0.001605813753803031, 0.0001559842453087506],
    [0.001559661122420557, 0.00015928125842445138],
    [0.0021289871851236427, 0.00016317128601887779],
    [0.0017530130024650842, 0.00025731447985278253],
    [0.0011007070529042728, 0.00010090888791834001],
    [0.0012090172094871771, 0.00013394119080470477],
    [0.001648936969040879, 9.993663236324522e-05],
    [0.0017209489436539936, 0.00010049517959384276],
    [0.0018836719120070068, 0.00014741960338955857],
    [0.002138300268623284, 0.00019051290448675263]], dtype=np.float32)

_MAG_BINS = np.arange(4.0, 15.0, 0.5, dtype=np.float32)

F32_ONE_BITS = 0x3F800000
BIG_BITS = int(np.float32(np.finfo(np.float32).max).view(np.int32))


def _f2b(x):
    return jax.lax.bitcast_convert_type(x, jnp.int32)


def _b2f(x):
    return jax.lax.bitcast_convert_type(x, jnp.float32)


def _select_kth(bits, k):
    r = jnp.zeros_like(k)
    for b in range(29, -1, -1):
        t = r | (1 << b)
        cnt = jnp.sum((bits < t).astype(jnp.int32), axis=-1, keepdims=True)
        r = jnp.where(cnt <= k, t, r)
    return r


def _stats_kernel(t_ref, m_ref, e_ref, mask_ref, kbits_ref, n2_ref):
    t = t_ref[...]
    m = m_ref[...]
    e = e_ref[...]
    msk = mask_ref[...] != 0
    inf = jnp.float32(np.inf)
    finite = (jnp.abs(t) < inf) | (jnp.abs(m) < inf) | (jnp.abs(e) < inf)
    valid0 = msk & finite & (e > 0.0)
    n0 = jnp.sum(valid0.astype(jnp.int32), axis=-1, keepdims=True)
    kmed = jnp.clip((n0 - 1) // 2, 0, L - 1)

    ebits = jnp.where(valid0, _f2b(e), F32_ONE_BITS)
    med = _b2f(_select_kth(ebits, kmed))
    a = jnp.abs(e - med)
    abits = jnp.where(valid0, _f2b(a), F32_ONE_BITS)
    mad = _b2f(_select_kth(abits, kmed))
    rel = (e > med + 5.0 * 1.4826 * mad) & valid0 & (mad > 0)

    s = jnp.zeros(t.shape, jnp.int32)
    for bk in _MAG_BINS:
        s = s + (float(bk) < m).astype(jnp.int32)
    idx = jnp.maximum(s - 1, 0)
    thr_t = _ERR_TABLE[:, 0] + np.float32(3.0 * 1.4826) * _ERR_TABLE[:, 1]
    thr = jnp.full(t.shape, float(thr_t[0]), jnp.float32)
    for kk in range(1, len(thr_t)):
        thr = jnp.where(idx == kk, float(thr_t[kk]), thr)
    magbin = e > thr

    valid1 = valid0 & ~(rel & magbin)
    n1 = jnp.sum(valid1.astype(jnp.int32), axis=-1, keepdims=True)
    mbits = jnp.where(valid1, _f2b(m), F32_ONE_BITS)
    nm1 = jnp.maximum(n1 - 1, 0).astype(jnp.float32)

    qv = []
    for qq in (0.25, 0.75):
        pos = qq * nm1
        fl = jnp.floor(pos)
        lo = jnp.clip(fl.astype(jnp.int32), 0, L - 1)
        hi = jnp.clip(jnp.ceil(pos).astype(jnp.int32), 0, L - 1)
        frac = pos - fl
        vlo_b = _select_kth(mbits, lo)
        gt = jnp.where(mbits > vlo_b, mbits, F32_ONE_BITS)
        mingt = jnp.min(gt, axis=-1, keepdims=True)
        cnt_le = jnp.sum((mbits <= vlo_b).astype(jnp.int32), axis=-1,
                         keepdims=True)
        nxt_b = jnp.where(cnt_le >= lo + 2, vlo_b, mingt)
        vhi_b = jnp.where(hi == lo, vlo_b, nxt_b)
        vlo = _b2f(vlo_b)
        vhi = _b2f(vhi_b)
        qv.append(vlo + (vhi - vlo) * frac)
    q1, q3 = qv
    iqr = jnp.maximum(q3 - q1, 0.015)
    bright = (m < q1 - 3.0 * iqr) & valid1
    faint = (m > q3 + 3.0 * iqr) & valid1
    outm = bright | faint
    pardon = jnp.sum(outm.astype(jnp.int32), axis=-1, keepdims=True) > 1
    outm = outm & ~pardon
    valid2 = valid1 & ~outm
    n2 = jnp.sum(valid2.astype(jnp.int32), axis=-1, keepdims=True)

    kbits_ref[...] = jnp.where(valid2, _f2b(t), F32_ONE_BITS)
    n2_ref[...] = n2


def _stats(t, m, e, mask):
    nb = t.shape[0]
    grid = (nb // BR,)
    spec = pl.BlockSpec((BR, L), lambda i: (i, 0))
    nspec = pl.BlockSpec((BR, 1), lambda i: (i, 0))
    return pl.pallas_call(
        _stats_kernel,
        grid=grid,
        in_specs=[spec, spec, spec, spec],
        out_specs=[spec, nspec],
        out_shape=[
            jax.ShapeDtypeStruct((nb, L), jnp.int32),
            jax.ShapeDtypeStruct((nb, 1), jnp.int32),
        ],
    )(t, m, e, mask.astype(jnp.int32))


NW = 32
ROWS_PER_W = B // NW
NV = L // 16
NBINS = 1024
NBV = NBINS // 16


def _sc_sort(kbits, light_curve, row_base, nrows):
    mesh = plsc.VectorSubcoreMesh(core_axis_name="c", subcore_axis_name="s")
    cp = dataclasses.replace(pltpu.CompilerParams(), needs_layout_passes=False,
                             use_tc_tiling_on_sc=False)
    rpw = nrows // NW

    def run(kb, lc):
        @pl.kernel(
            out_type=jax.ShapeDtypeStruct((nrows, 3, L), jnp.float32),
            mesh=mesh,
            compiler_params=cp,
            scratch_types=[
                pltpu.VMEM((L,), jnp.int32),
                pltpu.VMEM((L,), jnp.int32),
                pltpu.VMEM((L,), jnp.int32),
                pltpu.VMEM((L,), jnp.int32),
                pltpu.VMEM((L,), jnp.int32),
                pltpu.VMEM((L,), jnp.int32),
                pltpu.VMEM((NBINS,), jnp.int32),
                pltpu.VMEM((NBINS,), jnp.int32),
                pltpu.VMEM((NBINS,), jnp.int32),
                pltpu.VMEM((L,), jnp.float32),
                pltpu.VMEM((L,), jnp.float32),
                pltpu.VMEM((L,), jnp.float32),
                pltpu.VMEM((L,), jnp.float32),
                pltpu.VMEM((L,), jnp.float32),
                pltpu.VMEM((L,), jnp.float32),
                pltpu.VMEM((L,), jnp.float32),
                pltpu.VMEM((L,), jnp.float32),
                pltpu.VMEM((L,), jnp.float32),
                pltpu.VMEM((L,), jnp.float32),
                pltpu.VMEM((L,), jnp.float32),
                pltpu.VMEM((L,), jnp.float32),
                pltpu.SemaphoreType.DMA,
                pltpu.SemaphoreType.DMA,
                pltpu.SemaphoreType.DMA,
                pltpu.SemaphoreType.DMA,
                pltpu.SemaphoreType.DMA,
                pltpu.SemaphoreType.DMA,
            ],
        )
        def k(kb_hbm, lc_hbm, out_hbm, kin0, kin1, u_b, u_c, ix_a, ix_b,
              h_a, h_b, offs,
              ca0, ca1, ca2, cb0, cb1, cb2,
              oa0, oa1, oa2, ob0, ob1, ob2,
              sk0, sk1, si0, si1, so0, so1):
            sem_ks = (sk0, sk1)
            sem_ins = (si0, si1)
            sem_outs = (so0, so1)
            kin = (kin0, kin1)
            chb = ((ca0, ca1, ca2), (cb0, cb1, cb2))
            cob = ((oa0, oa1, oa2), (ob0, ob1, ob2))
            wid = lax.axis_index("s") * 2 + lax.axis_index("c")
            row0 = wid * rpw
            iota = lax.iota(jnp.int32, 16)
            ones = jnp.ones((16,), jnp.int32)
            zeros = jnp.zeros((16,), jnp.int32)

            def zero(h):
                @pl.loop(0, NBV)
                def _(i):
                    h[pl.ds(i * 16, 16)] = zeros

            def prefix(h):
                def pf(i, carry):
                    sl = pl.ds(i * 16, 16)
                    v = h[sl]
                    h[sl] = zeros
                    inc = plsc.cumsum(v)
                    offs[sl] = inc - v + (carry - 1)
                    return carry + jnp.sum(v)

                lax.fori_loop(0, NBV, pf, jnp.int32(0))

            def body(j, bank):
                r = row0 + j
                rg = row_base + r
                @pl.when(j < rpw - 1)
                def _():
                    nb = chb[1 - bank]
                    pltpu.async_copy(kb_hbm.at[r + 1], kin[1 - bank],
                                     sem_ks[1 - bank])
                    pltpu.async_copy(lc_hbm.at[rg + 1, 0], nb[0],
                                     sem_ins[1 - bank])
                    pltpu.async_copy(lc_hbm.at[rg + 1, 1], nb[1],
                                     sem_ins[1 - bank])
                    pltpu.async_copy(lc_hbm.at[rg + 1, 2], nb[2],
                                     sem_ins[1 - bank])

                u_in = kin[bank]
                pltpu.make_async_copy(kb_hbm.at[r], u_in, sem_ks[bank]).wait()

                @pl.loop(0, NV)
                def _(i):
                    d = u_in[pl.ds(i * 16, 16)] & (NBINS - 1)
                    plsc.addupdate_scatter(h_a, [d], ones)

                prefix(h_a)

                @pl.loop(0, NV)
                def _(i):
                    u = u_in[pl.ds(i * 16, 16)]
                    d = u & (NBINS - 1)
                    cnt, is_last = plsc.scan_count(d)
                    base = plsc.load_gather(offs, [d])
                    pos = base + cnt
                    plsc.store_scatter(u_b, [pos], u)
                    plsc.store_scatter(ix_a, [pos], iota + i * 16)
                    d1 = (u >> 10) & (NBINS - 1)
                    plsc.addupdate_scatter(h_b, [d1], ones)
                    plsc.addupdate_scatter(offs, [d], cnt, mask=is_last)

                prefix(h_b)

                @pl.loop(0, NV)
                def _(i):
                    sl = pl.ds(i * 16, 16)
                    u = u_b[sl]
                    d = (u >> 10) & (NBINS - 1)
                    cnt, is_last = plsc.scan_count(d)
                    base = plsc.load_gather(offs, [d])
                    pos = base + cnt
                    plsc.store_scatter(u_c, [pos], u)
                    plsc.store_scatter(ix_b, [pos], ix_a[sl])
                    d2 = u >> 20
                    plsc.addupdate_scatter(h_a, [d2], ones)
                    plsc.addupdate_scatter(offs, [d], cnt, mask=is_last)

                ch = chb[bank]
                co = cob[bank]
                sin = sem_ins[bank]
                pltpu.make_async_copy(lc_hbm.at[rg, 0], ch[0], sin).wait()
                pltpu.make_async_copy(lc_hbm.at[rg, 1], ch[1], sin).wait()
                pltpu.make_async_copy(lc_hbm.at[rg, 2], ch[2], sin).wait()

                @pl.when(j >= 2)
                def _():
                    for c in range(3):
                        pltpu.make_async_copy(
                            co[c], out_hbm.at[r - 2, c], sem_outs[bank]).wait()

                prefix(h_a)

                @pl.loop(0, NV)
                def _(i):
                    sl = pl.ds(i * 16, 16)
                    u = u_c[sl]
                    d = u >> 20
                    cnt, is_last = plsc.scan_count(d)
                    base = plsc.load_gather(offs, [d])
                    pos = base + cnt
                    iv = ix_b[sl]
                    for c in range(3):
                        val = plsc.load_gather(ch[c], [iv])
                        plsc.store_scatter(co[c], [pos], val)
                    plsc.addupdate_scatter(offs, [d], cnt, mask=is_last)

                for c in range(3):
                    pltpu.async_copy(co[c], out_hbm.at[r, c], sem_outs[bank])

            zero(h_a)
            zero(h_b)
            pltpu.async_copy(kb_hbm.at[row0], kin0, sk0)
            pltpu.async_copy(lc_hbm.at[row_base + row0, 0], ca0, si0)
            pltpu.async_copy(lc_hbm.at[row_base + row0, 1], ca1, si0)
            pltpu.async_copy(lc_hbm.at[row_base + row0, 2], ca2, si0)

            @pl.loop(0, rpw // 2)
            def _(jj):
                body(2 * jj, 0)
                body(2 * jj + 1, 1)

            for (j, bank) in ((rpw - 2, 0), (rpw - 1, 1)):
                for c in range(3):
                    pltpu.make_async_copy(
                        cob[bank][c], out_hbm.at[row0 + j, c],
                        sem_outs[bank]).wait()

        return k(kb, lc)

    return run(kbits, light_curve)


NCHUNK = 2


def kernel(light_curve, non_padded_mask):
    bh = B // NCHUNK
    outs, vouts = [], []
    for ci in range(NCHUNK):
        sl = slice(ci * bh, (ci + 1) * bh)
        t = light_curve[sl, 0, :]
        m = light_curve[sl, 1, :]
        e = light_curve[sl, 2, :]
        kbits, n2 = _stats(t, m, e, non_padded_mask[sl])
        vouts.append(jnp.arange(L, dtype=jnp.int32)[None, :] < n2)
        outs.append(_sc_sort(kbits, light_curve, ci * bh, bh))
    lc_sorted = jnp.concatenate(outs, axis=0)
    valid_sorted = jnp.concatenate(vouts, axis=0)
    return lc_sorted, valid_sorted


# speedup vs baseline: 1.0096x; 1.0096x over previous
import dataclasses

import numpy as np
import jax
import jax.numpy as jnp
from jax import lax
from jax.experimental import pallas as pl
from jax.experimental.pallas import tpu as pltpu
from jax.experimental.pallas import tpu_sc as plsc

B, L = 4096, 2048
BR = 64

_ERR_TABLE = np.array([
    [0.0072346226959187085, 0.000556539138304629],
    [0.007277966762969894, 0.0006341286519172679],
    [0.00751161242916645, 0.0003549648152118088],
    [0.003841922673385825, 0.0009420231103613149],
    [0.0028823807111780318, 0.00028250797805103186],
    [0.0025199820839086813, 0.0003671334743430223],
    [0.002398287605344948, 0.0002502935178378177],
    [0.0022981766524078407, 0.0002689046639815398],
    [0.0023829754751450204, 0.00029003694763449195],
    [0.002103783741402776, 0.00022834709513293414],
    [0.001680953479241577, 0.00017038730690437185],
    [0.0015808278649738481, 0.0001357746865960827],
    [0.001605813753803031, 0.0001559842453087506],
    [0.001559661122420557, 0.00015928125842445138],
    [0.0021289871851236427, 0.00016317128601887779],
    [0.0017530130024650842, 0.00025731447985278253],
    [0.0011007070529042728, 0.00010090888791834001],
    [0.0012090172094871771, 0.00013394119080470477],
    [0.001648936969040879, 9.993663236324522e-05],
    [0.0017209489436539936, 0.00010049517959384276],
    [0.0018836719120070068, 0.00014741960338955857],
    [0.002138300268623284, 0.00019051290448675263]], dtype=np.float32)

_MAG_BINS = np.arange(4.0, 15.0, 0.5, dtype=np.float32)

F32_ONE_BITS = 0x3F800000
BIG_BITS = int(np.float32(np.finfo(np.float32).max).view(np.int32))


def _f2b(x):
    return jax.lax.bitcast_convert_type(x, jnp.int32)


def _b2f(x):
    return jax.lax.bitcast_convert_type(x, jnp.float32)


def _select_kth(bits, k):
    r = jnp.zeros_like(k)
    for b in range(29, -1, -1):
        t = r | (1 << b)
        cnt = jnp.sum((bits < t).astype(jnp.int32), axis=-1, keepdims=True)
        r = jnp.where(cnt <= k, t, r)
    return r


def _stats_kernel(t_ref, m_ref, e_ref, mask_ref, kbits_ref, n2_ref):
    t = t_ref[...]
    m = m_ref[...]
    e = e_ref[...]
    msk = mask_ref[...] != 0
    inf = jnp.float32(np.inf)
    finite = (jnp.abs(t) < inf) | (jnp.abs(m) < inf) | (jnp.abs(e) < inf)
    valid0 = msk & finite & (e > 0.0)
    n0 = jnp.sum(valid0.astype(jnp.int32), axis=-1, keepdims=True)
    kmed = jnp.clip((n0 - 1) // 2, 0, L - 1)

    ebits = jnp.where(valid0, _f2b(e), F32_ONE_BITS)
    med = _b2f(_select_kth(ebits, kmed))
    a = jnp.abs(e - med)
    abits = jnp.where(valid0, _f2b(a), F32_ONE_BITS)
    mad = _b2f(_select_kth(abits, kmed))
    rel = (e > med + 5.0 * 1.4826 * mad) & valid0 & (mad > 0)

    s = jnp.zeros(t.shape, jnp.int32)
    for bk in _MAG_BINS:
        s = s + (float(bk) < m).astype(jnp.int32)
    idx = jnp.maximum(s - 1, 0)
    thr_t = _ERR_TABLE[:, 0] + np.float32(3.0 * 1.4826) * _ERR_TABLE[:, 1]
    thr = jnp.full(t.shape, float(thr_t[0]), jnp.float32)
    for kk in range(1, len(thr_t)):
        thr = jnp.where(idx == kk, float(thr_t[kk]), thr)
    magbin = e > thr

    valid1 = valid0 & ~(rel & magbin)
    n1 = jnp.sum(valid1.astype(jnp.int32), axis=-1, keepdims=True)
    mbits = jnp.where(valid1, _f2b(m), F32_ONE_BITS)
    nm1 = jnp.maximum(n1 - 1, 0).astype(jnp.float32)

    qv = []
    for qq in (0.25, 0.75):
        pos = qq * nm1
        fl = jnp.floor(pos)
        lo = jnp.clip(fl.astype(jnp.int32), 0, L - 1)
        hi = jnp.clip(jnp.ceil(pos).astype(jnp.int32), 0, L - 1)
        frac = pos - fl
        vlo_b = _select_kth(mbits, lo)
        gt = jnp.where(mbits > vlo_b, mbits, F32_ONE_BITS)
        mingt = jnp.min(gt, axis=-1, keepdims=True)
        cnt_le = jnp.sum((mbits <= vlo_b).astype(jnp.int32), axis=-1,
                         keepdims=True)
        nxt_b = jnp.where(cnt_le >= lo + 2, vlo_b, mingt)
        vhi_b = jnp.where(hi == lo, vlo_b, nxt_b)
        vlo = _b2f(vlo_b)
        vhi = _b2f(vhi_b)
        qv.append(vlo + (vhi - vlo) * frac)
    q1, q3 = qv
    iqr = jnp.maximum(q3 - q1, 0.015)
    bright = (m < q1 - 3.0 * iqr) & valid1
    faint = (m > q3 + 3.0 * iqr) & valid1
    outm = bright | faint
    pardon = jnp.sum(outm.astype(jnp.int32), axis=-1, keepdims=True) > 1
    outm = outm & ~pardon
    valid2 = valid1 & ~outm
    n2 = jnp.sum(valid2.astype(jnp.int32), axis=-1, keepdims=True)

    kbits_ref[...] = jnp.where(valid2, _f2b(t), F32_ONE_BITS)
    n2_ref[...] = n2


def _stats(t, m, e, mask):
    nb = t.shape[0]
    grid = (nb // BR,)
    spec = pl.BlockSpec((BR, L), lambda i: (i, 0))
    nspec = pl.BlockSpec((BR, 1), lambda i: (i, 0))
    return pl.pallas_call(
        _stats_kernel,
        grid=grid,
        in_specs=[spec, spec, spec, spec],
        out_specs=[spec, nspec],
        out_shape=[
            jax.ShapeDtypeStruct((nb, L), jnp.int32),
            jax.ShapeDtypeStruct((nb, 1), jnp.int32),
        ],
    )(t, m, e, mask.astype(jnp.int32))


NW = 32
ROWS_PER_W = B // NW
NV = L // 16
NBINS = 1024
NBV = NBINS // 16


def _sc_sort(kbits, light_curve, row_base, nrows):
    mesh = plsc.VectorSubcoreMesh(core_axis_name="c", subcore_axis_name="s")
    cp = dataclasses.replace(pltpu.CompilerParams(), needs_layout_passes=False,
                             use_tc_tiling_on_sc=False)
    rpw = nrows // NW

    def run(kb, lc):
        vmem_i32 = lambda n: pltpu.VMEM((n,), jnp.int32)
        vmem_f32 = lambda n: pltpu.VMEM((n,), jnp.float32)
        scratch = (
            [vmem_i32(L) for _ in range(4)]
            + [vmem_i32(L) for _ in range(4)]
            + [vmem_i32(NBINS) for _ in range(6)]
            + [vmem_f32(L) for _ in range(12)]
            + [vmem_f32(L) for _ in range(12)]
            + [pltpu.SemaphoreType.DMA] * 6
        )

        @pl.kernel(
            out_type=jax.ShapeDtypeStruct((nrows, 3, L), jnp.float32),
            mesh=mesh,
            compiler_params=cp,
            scratch_types=scratch,
        )
        def k(kb_hbm, lc_hbm, out_hbm, *refs):
            kin = refs[0:4]
            ix_a = refs[4:6]
            ix_b = refs[6:8]
            h_a = refs[8:10]
            h_b = refs[10:12]
            offs = refs[12:14]
            ch = tuple(refs[14 + 3 * s:17 + 3 * s] for s in range(4))
            co = tuple(refs[26 + 3 * s:29 + 3 * s] for s in range(4))
            sk = refs[38:40]
            si = refs[40:42]
            so = refs[42:44]
            wid = lax.axis_index("s") * 2 + lax.axis_index("c")
            row0 = wid * rpw
            iota = lax.iota(jnp.int32, 16)
            ones = jnp.ones((16,), jnp.int32)
            zeros = jnp.zeros((16,), jnp.int32)

            def zero(h):
                @pl.loop(0, NBV)
                def _(i):
                    h[pl.ds(i * 16, 16)] = zeros

            def prefix2(h0, h1):
                def pf(i, cc):
                    c0, c1 = cc
                    sl = pl.ds(i * 16, 16)
                    v0 = h0[sl]
                    h0[sl] = zeros
                    offs[0][sl] = plsc.cumsum(v0) - v0 + (c0 - 1)
                    v1 = h1[sl]
                    h1[sl] = zeros
                    offs[1][sl] = plsc.cumsum(v1) - v1 + (c1 - 1)
                    return (c0 + jnp.sum(v0), c1 + jnp.sum(v1))

                lax.fori_loop(0, NBV, pf, (jnp.int32(0), jnp.int32(0)))

            def issue_pair(j, bs):
                for s in range(2):
                    r = row0 + j + s
                    rg = row_base + r
                    slot = 2 * bs + s
                    pltpu.async_copy(kb_hbm.at[r], kin[slot], sk[bs])
                    for c in range(3):
                        pltpu.async_copy(lc_hbm.at[rg, c], ch[slot][c], si[bs])

            def body_pair(j, bs):
                slots = (2 * bs, 2 * bs + 1)
                @pl.when(j < rpw - 2)
                def _():
                    issue_pair(j + 2, 1 - bs)

                for s in range(2):
                    pltpu.make_async_copy(kb_hbm.at[row0 + j + s],
                                          kin[slots[s]], sk[bs]).wait()

                @pl.loop(0, NV)
                def _(i):
                    sl = pl.ds(i * 16, 16)
                    for s in range(2):
                        d = kin[slots[s]][sl] & (NBINS - 1)
                        plsc.addupdate_scatter(h_a[s], [d], ones)

                prefix2(h_a[0], h_a[1])

                @pl.loop(0, NV)
                def _(i):
                    sl = pl.ds(i * 16, 16)
                    for s in range(2):
                        u = kin[slots[s]][sl]
                        d = u & (NBINS - 1)
                        cnt, is_last = plsc.scan_count(d)
                        base = plsc.load_gather(offs[s], [d])
                        pos = base + cnt
                        plsc.store_scatter(ix_a[s], [pos], iota + i * 16)
                        d1 = (u >> 10) & (NBINS - 1)
                        plsc.addupdate_scatter(h_b[s], [d1], ones)
                        plsc.addupdate_scatter(offs[s], [d], cnt, mask=is_last)

                prefix2(h_b[0], h_b[1])

                @pl.loop(0, NV)
                def _(i):
                    sl = pl.ds(i * 16, 16)
                    for s in range(2):
                        iv = ix_a[s][sl]
                        u = plsc.load_gather(kin[slots[s]], [iv])
                        d = (u >> 10) & (NBINS - 1)
                        cnt, is_last = plsc.scan_count(d)
                        base = plsc.load_gather(offs[s], [d])
                        pos = base + cnt
                        plsc.store_scatter(ix_b[s], [pos], iv)
                        d2 = u >> 20
                        plsc.addupdate_scatter(h_a[s], [d2], ones)
                        plsc.addupdate_scatter(offs[s], [d], cnt, mask=is_last)

                for s in range(2):
                    rg = row_base + row0 + j + s
                    for c in range(3):
                        pltpu.make_async_copy(lc_hbm.at[rg, c],
                                              ch[slots[s]][c], si[bs]).wait()

                @pl.when(j >= 4)
                def _():
                    for s in range(2):
                        for c in range(3):
                            pltpu.make_async_copy(
                                co[slots[s]][c],
                                out_hbm.at[row0 + j - 4 + s, c],
                                so[bs]).wait()

                prefix2(h_a[0], h_a[1])

                @pl.loop(0, NV)
                def _(i):
                    sl = pl.ds(i * 16, 16)
                    for s in range(2):
                        iv = ix_b[s][sl]
                        u = plsc.load_gather(kin[slots[s]], [iv])
                        d = u >> 20
                        cnt, is_last = plsc.scan_count(d)
                        base = plsc.load_gather(offs[s], [d])
                        pos = base + cnt
                        for c in range(3):
                            val = plsc.load_gather(ch[slots[s]][c], [iv])
                            plsc.store_scatter(co[slots[s]][c], [pos], val)
                        plsc.addupdate_scatter(offs[s], [d], cnt, mask=is_last)

                for s in range(2):
                    for c in range(3):
                        pltpu.async_copy(co[slots[s]][c],
                                         out_hbm.at[row0 + j + s, c], so[bs])

            for s in range(2):
                zero(h_a[s])
                zero(h_b[s])
            issue_pair(0, 0)

            @pl.loop(0, rpw // 4)
            def _(pp):
                body_pair(4 * pp, 0)
                body_pair(4 * pp + 2, 1)

            for (j, bs) in ((rpw - 4, 0), (rpw - 2, 1)):
                for s in range(2):
                    for c in range(3):
                        pltpu.make_async_copy(
                            co[2 * bs + s][c],
                            out_hbm.at[row0 + j + s, c], so[bs]).wait()

        return k(kb, lc)

    return run(kbits, light_curve)


NCHUNK = 1


def kernel(light_curve, non_padded_mask):
    bh = B // NCHUNK
    outs, vouts = [], []
    for ci in range(NCHUNK):
        sl = slice(ci * bh, (ci + 1) * bh)
        t = light_curve[sl, 0, :]
        m = light_curve[sl, 1, :]
        e = light_curve[sl, 2, :]
        kbits, n2 = _stats(t, m, e, non_padded_mask[sl])
        vouts.append(jnp.arange(L, dtype=jnp.int32)[None, :] < n2)
        outs.append(_sc_sort(kbits, light_curve, ci * bh, bh))
    lc_sorted = jnp.concatenate(outs, axis=0)
    valid_sorted = jnp.concatenate(vouts, axis=0)
    return lc_sorted, valid_sorted


# speedup vs baseline: 1.1305x; 1.1198x over previous
import dataclasses

import numpy as np
import jax
import jax.numpy as jnp
from jax import lax
from jax.experimental import pallas as pl
from jax.experimental.pallas import tpu as pltpu
from jax.experimental.pallas import tpu_sc as plsc

B, L = 4096, 2048
BR = 128

_ERR_TABLE = np.array([
    [0.0072346226959187085, 0.000556539138304629],
    [0.007277966762969894, 0.0006341286519172679],
    [0.00751161242916645, 0.0003549648152118088],
    [0.003841922673385825, 0.0009420231103613149],
    [0.0028823807111780318, 0.00028250797805103186],
    [0.0025199820839086813, 0.0003671334743430223],
    [0.002398287605344948, 0.0002502935178378177],
    [0.0022981766524078407, 0.0002689046639815398],
    [0.0023829754751450204, 0.00029003694763449195],
    [0.002103783741402776, 0.00022834709513293414],
    [0.001680953479241577, 0.00017038730690437185],
    [0.0015808278649738481, 0.0001357746865960827],
    [0.001605813753803031, 0.0001559842453087506],
    [0.001559661122420557, 0.00015928125842445138],
    [0.0021289871851236427, 0.00016317128601887779],
    [0.0017530130024650842, 0.00025731447985278253],
    [0.0011007070529042728, 0.00010090888791834001],
    [0.0012090172094871771, 0.00013394119080470477],
    [0.001648936969040879, 9.993663236324522e-05],
    [0.0017209489436539936, 0.00010049517959384276],
    [0.0018836719120070068, 0.00014741960338955857],
    [0.002138300268623284, 0.00019051290448675263]], dtype=np.float32)

_MAG_BINS = np.arange(4.0, 15.0, 0.5, dtype=np.float32)

F32_ONE_BITS = 0x3F800000
BIG_BITS = int(np.float32(np.finfo(np.float32).max).view(np.int32))


def _f2b(x):
    return jax.lax.bitcast_convert_type(x, jnp.int32)


def _b2f(x):
    return jax.lax.bitcast_convert_type(x, jnp.float32)


def _select_kth(bits, k):
    r = jnp.zeros_like(k)
    for b in range(29, -1, -1):
        t = r | (1 << b)
        cnt = jnp.sum((bits < t).astype(jnp.int32), axis=-1, keepdims=True)
        r = jnp.where(cnt <= k, t, r)
    return r


def _stats_kernel(t_ref, m_ref, e_ref, mask_ref, kbits_ref, n2_ref):
    t = t_ref[...]
    m = m_ref[...]
    e = e_ref[...]
    msk = mask_ref[...] != 0
    inf = jnp.float32(np.inf)
    finite = (jnp.abs(t) < inf) | (jnp.abs(m) < inf) | (jnp.abs(e) < inf)
    valid0 = msk & finite & (e > 0.0)
    n0 = jnp.sum(valid0.astype(jnp.int32), axis=-1, keepdims=True)
    kmed = jnp.clip((n0 - 1) // 2, 0, L - 1)

    ebits = jnp.where(valid0, _f2b(e), F32_ONE_BITS)
    med = _b2f(_select_kth(ebits, kmed))
    a = jnp.abs(e - med)
    abits = jnp.where(valid0, _f2b(a), F32_ONE_BITS)
    mad = _b2f(_select_kth(abits, kmed))
    rel = (e > med + 5.0 * 1.4826 * mad) & valid0 & (mad > 0)

    s = jnp.zeros(t.shape, jnp.int32)
    for bk in _MAG_BINS:
        s = s + (float(bk) < m).astype(jnp.int32)
    idx = jnp.maximum(s - 1, 0)
    thr_t = _ERR_TABLE[:, 0] + np.float32(3.0 * 1.4826) * _ERR_TABLE[:, 1]
    thr = jnp.full(t.shape, float(thr_t[0]), jnp.float32)
    for kk in range(1, len(thr_t)):
        thr = jnp.where(idx == kk, float(thr_t[kk]), thr)
    magbin = e > thr

    valid1 = valid0 & ~(rel & magbin)
    n1 = jnp.sum(valid1.astype(jnp.int32), axis=-1, keepdims=True)
    mbits = jnp.where(valid1, _f2b(m), F32_ONE_BITS)
    nm1 = jnp.maximum(n1 - 1, 0).astype(jnp.float32)

    qv = []
    for qq in (0.25, 0.75):
        pos = qq * nm1
        fl = jnp.floor(pos)
        lo = jnp.clip(fl.astype(jnp.int32), 0, L - 1)
        hi = jnp.clip(jnp.ceil(pos).astype(jnp.int32), 0, L - 1)
        frac = pos - fl
        vlo_b = _select_kth(mbits, lo)
        gt = jnp.where(mbits > vlo_b, mbits, F32_ONE_BITS)
        mingt = jnp.min(gt, axis=-1, keepdims=True)
        cnt_le = jnp.sum((mbits <= vlo_b).astype(jnp.int32), axis=-1,
                         keepdims=True)
        nxt_b = jnp.where(cnt_le >= lo + 2, vlo_b, mingt)
        vhi_b = jnp.where(hi == lo, vlo_b, nxt_b)
        vlo = _b2f(vlo_b)
        vhi = _b2f(vhi_b)
        qv.append(vlo + (vhi - vlo) * frac)
    q1, q3 = qv
    iqr = jnp.maximum(q3 - q1, 0.015)
    bright = (m < q1 - 3.0 * iqr) & valid1
    faint = (m > q3 + 3.0 * iqr) & valid1
    outm = bright | faint
    pardon = jnp.sum(outm.astype(jnp.int32), axis=-1, keepdims=True) > 1
    outm = outm & ~pardon
    valid2 = valid1 & ~outm
    n2 = jnp.sum(valid2.astype(jnp.int32), axis=-1, keepdims=True)

    kbits_ref[...] = jnp.where(valid2, _f2b(t), F32_ONE_BITS)
    n2_ref[...] = n2


def _stats(t, m, e, mask):
    nb = t.shape[0]
    grid = (nb // BR,)
    spec = pl.BlockSpec((BR, L), lambda i: (i, 0))
    nspec = pl.BlockSpec((BR, 1), lambda i: (i, 0))
    return pl.pallas_call(
        _stats_kernel,
        grid=grid,
        in_specs=[spec, spec, spec, spec],
        out_specs=[spec, nspec],
        out_shape=[
            jax.ShapeDtypeStruct((nb, L), jnp.int32),
            jax.ShapeDtypeStruct((nb, 1), jnp.int32),
        ],
    )(t, m, e, mask.astype(jnp.int32))


NW = 32
ROWS_PER_W = B // NW
NV = L // 16
NBINS = 1024
NBV = NBINS // 16


def _sc_sort(kbits, light_curve, row_base, nrows):
    mesh = plsc.VectorSubcoreMesh(core_axis_name="c", subcore_axis_name="s")
    cp = dataclasses.replace(pltpu.CompilerParams(), needs_layout_passes=False,
                             use_tc_tiling_on_sc=False)
    rpw = nrows // NW

    def run(kb, lc):
        @pl.kernel(
            out_type=jax.ShapeDtypeStruct((nrows, 3, L), jnp.float32),
            mesh=mesh,
            compiler_params=cp,
            scratch_types=[
                pltpu.VMEM((L,), jnp.int32),
                pltpu.VMEM((L,), jnp.int32),
                pltpu.VMEM((L,), jnp.int32),
                pltpu.VMEM((L,), jnp.int32),
                pltpu.VMEM((L,), jnp.int32),
                pltpu.VMEM((L,), jnp.int32),
                pltpu.VMEM((NBINS,), jnp.int32),
                pltpu.VMEM((NBINS,), jnp.int32),
                pltpu.VMEM((NBINS,), jnp.int32),
                pltpu.VMEM((L,), jnp.float32),
                pltpu.VMEM((L,), jnp.float32),
                pltpu.VMEM((L,), jnp.float32),
                pltpu.VMEM((L,), jnp.float32),
                pltpu.VMEM((L,), jnp.float32),
                pltpu.VMEM((L,), jnp.float32),
                pltpu.VMEM((L,), jnp.float32),
                pltpu.VMEM((L,), jnp.float32),
                pltpu.VMEM((L,), jnp.float32),
                pltpu.VMEM((L,), jnp.float32),
                pltpu.VMEM((L,), jnp.float32),
                pltpu.VMEM((L,), jnp.float32),
                pltpu.SemaphoreType.DMA,
                pltpu.SemaphoreType.DMA,
                pltpu.SemaphoreType.DMA,
                pltpu.SemaphoreType.DMA,
                pltpu.SemaphoreType.DMA,
                pltpu.SemaphoreType.DMA,
            ],
        )
        def k(kb_hbm, lc_hbm, out_hbm, kin0, kin1, u_b, u_c, ix_a, ix_b,
              h_a, h_b, offs,
              ca0, ca1, ca2, cb0, cb1, cb2,
              oa0, oa1, oa2, ob0, ob1, ob2,
              sk0, sk1, si0, si1, so0, so1):
            sem_ks = (sk0, sk1)
            sem_ins = (si0, si1)
            sem_outs = (so0, so1)
            kin = (kin0, kin1)
            chb = ((ca0, ca1, ca2), (cb0, cb1, cb2))
            cob = ((oa0, oa1, oa2), (ob0, ob1, ob2))
            wid = lax.axis_index("s") * 2 + lax.axis_index("c")
            row0 = wid * rpw
            iota = lax.iota(jnp.int32, 16)
            ones = jnp.ones((16,), jnp.int32)
            zeros = jnp.zeros((16,), jnp.int32)

            def zero(h):
                @pl.loop(0, NBV)
                def _(i):
                    h[pl.ds(i * 16, 16)] = zeros

            def prefix(h):
                def pf(i, carry):
                    sl = pl.ds(i * 16, 16)
                    v = h[sl]
                    h[sl] = zeros
                    inc = plsc.cumsum(v)
                    offs[sl] = inc - v + (carry - 1)
                    return carry + jnp.sum(v)

                lax.fori_loop(0, NBV, pf, jnp.int32(0))

            def body(j, bank):
                r = row0 + j
                rg = row_base + r
                @pl.when(j < rpw - 1)
                def _():
                    nb = chb[1 - bank]
                    pltpu.async_copy(kb_hbm.at[r + 1], kin[1 - bank],
                                     sem_ks[1 - bank])
                    pltpu.async_copy(lc_hbm.at[rg + 1, 0], nb[0],
                                     sem_ins[1 - bank])
                    pltpu.async_copy(lc_hbm.at[rg + 1, 1], nb[1],
                                     sem_ins[1 - bank])
                    pltpu.async_copy(lc_hbm.at[rg + 1, 2], nb[2],
                                     sem_ins[1 - bank])

                u_in = kin[bank]
                pltpu.make_async_copy(kb_hbm.at[r], u_in, sem_ks[bank]).wait()

                @pl.loop(0, NV)
                def _(i):
                    d = u_in[pl.ds(i * 16, 16)] & (NBINS - 1)
                    plsc.addupdate_scatter(h_a, [d], ones)

                prefix(h_a)

                @pl.loop(0, NV)
                def _(i):
                    u = u_in[pl.ds(i * 16, 16)]
                    d = u & (NBINS - 1)
                    cnt, is_last = plsc.scan_count(d)
                    base = plsc.load_gather(offs, [d])
                    pos = base + cnt
                    plsc.store_scatter(u_b, [pos], u)
                    plsc.store_scatter(ix_a, [pos], iota + i * 16)
                    d1 = (u >> 10) & (NBINS - 1)
                    plsc.addupdate_scatter(h_b, [d1], ones)
                    plsc.addupdate_scatter(offs, [d], cnt, mask=is_last)

                prefix(h_b)

                @pl.loop(0, NV)
                def _(i):
                    sl = pl.ds(i * 16, 16)
                    u = u_b[sl]
                    d = (u >> 10) & (NBINS - 1)
                    cnt, is_last = plsc.scan_count(d)
                    base = plsc.load_gather(offs, [d])
                    pos = base + cnt
                    plsc.store_scatter(u_c, [pos], u)
                    plsc.store_scatter(ix_b, [pos], ix_a[sl])
                    d2 = u >> 20
                    plsc.addupdate_scatter(h_a, [d2], ones)
                    plsc.addupdate_scatter(offs, [d], cnt, mask=is_last)

                ch = chb[bank]
                co = cob[bank]
                sin = sem_ins[bank]
                pltpu.make_async_copy(lc_hbm.at[rg, 0], ch[0], sin).wait()
                pltpu.make_async_copy(lc_hbm.at[rg, 1], ch[1], sin).wait()
                pltpu.make_async_copy(lc_hbm.at[rg, 2], ch[2], sin).wait()

                @pl.when(j >= 2)
                def _():
                    for c in range(3):
                        pltpu.make_async_copy(
                            co[c], out_hbm.at[r - 2, c], sem_outs[bank]).wait()

                prefix(h_a)

                @pl.loop(0, NV)
                def _(i):
                    sl = pl.ds(i * 16, 16)
                    u = u_c[sl]
                    d = u >> 20
                    cnt, is_last = plsc.scan_count(d)
                    base = plsc.load_gather(offs, [d])
                    pos = base + cnt
                    iv = ix_b[sl]
                    for c in range(3):
                        val = plsc.load_gather(ch[c], [iv])
                        plsc.store_scatter(co[c], [pos], val)
                    plsc.addupdate_scatter(offs, [d], cnt, mask=is_last)

                for c in range(3):
                    pltpu.async_copy(co[c], out_hbm.at[r, c], sem_outs[bank])

            zero(h_a)
            zero(h_b)
            pltpu.async_copy(kb_hbm.at[row0], kin0, sk0)
            pltpu.async_copy(lc_hbm.at[row_base + row0, 0], ca0, si0)
            pltpu.async_copy(lc_hbm.at[row_base + row0, 1], ca1, si0)
            pltpu.async_copy(lc_hbm.at[row_base + row0, 2], ca2, si0)

            @pl.loop(0, rpw // 2)
            def _(jj):
                body(2 * jj, 0)
                body(2 * jj + 1, 1)

            for (j, bank) in ((rpw - 2, 0), (rpw - 1, 1)):
                for c in range(3):
                    pltpu.make_async_copy(
                        cob[bank][c], out_hbm.at[row0 + j, c],
                        sem_outs[bank]).wait()

        return k(kb, lc)

    return run(kbits, light_curve)


NCHUNK = 1


def kernel(light_curve, non_padded_mask):
    bh = B // NCHUNK
    outs, vouts = [], []
    for ci in range(NCHUNK):
        sl = slice(ci * bh, (ci + 1) * bh)
        t = light_curve[sl, 0, :]
        m = light_curve[sl, 1, :]
        e = light_curve[sl, 2, :]
        kbits, n2 = _stats(t, m, e, non_padded_mask[sl])
        vouts.append(jnp.arange(L, dtype=jnp.int32)[None, :] < n2)
        outs.append(_sc_sort(kbits, light_curve, ci * bh, bh))
    lc_sorted = jnp.concatenate(outs, axis=0)
    valid_sorted = jnp.concatenate(vouts, axis=0)
    return lc_sorted, valid_sorted
